# baseline (device time: 19519 ns/iter reference)
import jax
import jax.numpy as jnp
from jax import lax
from jax.experimental import pallas as pl
from jax.experimental.pallas import tpu as pltpu

N_DEV = 4


def kernel(x, Wq, Wo, K_ext, V_ext):
    B, Sq, D = x.shape
    Dq = Wq.shape[1]
    Dh = K_ext.shape[3]
    Skv = K_ext.shape[1]
    Hq_local = Dq // Dh
    GQA = 4
    Hkv_local = Hq_local // GQA
    Hkv = K_ext.shape[2]
    Dout = Wo.shape[1]
    M = B * Sq

    Kt = jnp.transpose(K_ext, (0, 2, 3, 1))
    Vt = jnp.transpose(V_ext, (0, 2, 3, 1))

    def body(x_ref, wq_hbm, wo_hbm, kt_hbm, vt_hbm, out_ref,
             wq_vmem, wo_vmem, kt_vmem, vt_vmem,
             load_sems, comm_ref, send_sems, recv_sems):
        my_i = lax.axis_index("i")

        cp_wq = pltpu.make_async_copy(wq_hbm, wq_vmem, load_sems.at[0])
        cp_wo = pltpu.make_async_copy(wo_hbm, wo_vmem, load_sems.at[1])
        cp_k = pltpu.make_async_copy(
            kt_hbm.at[:, pl.ds(2 * my_i, Hkv_local)], kt_vmem,
            load_sems.at[2])
        cp_v = pltpu.make_async_copy(
            vt_hbm.at[:, pl.ds(2 * my_i, Hkv_local)], vt_vmem,
            load_sems.at[3])
        cp_wq.start()
        cp_k.start()
        cp_v.start()
        cp_wo.start()

        barrier_sem = pltpu.get_barrier_semaphore()
        for d in range(1, N_DEV):
            peer = lax.rem(my_i + d, N_DEV)
            pl.semaphore_signal(
                barrier_sem, inc=1,
                device_id=(peer,), device_id_type=pl.DeviceIdType.MESH,
            )
        pl.semaphore_wait(barrier_sem, N_DEV - 1)

        xv = x_ref[:].reshape(M, D).astype(jnp.bfloat16)
        cp_wq.wait()
        wq = wq_vmem[:].astype(jnp.bfloat16)
        q2 = (lax.dot(xv, wq, preferred_element_type=jnp.float32)
              * 0.125).astype(jnp.bfloat16)

        cp_k.wait()
        cp_v.wait()
        batch_rows = []
        for b in range(B):
            qb = q2[b * Sq:(b + 1) * Sq, :]
            heads = []
            for g in range(Hkv_local):
                kbt = kt_vmem[b, g].astype(jnp.bfloat16)
                vbt = vt_vmem[b, g].astype(jnp.bfloat16)
                qg = jnp.concatenate(
                    [qb[:, (g * GQA + hh) * Dh:(g * GQA + hh + 1) * Dh]
                     for hh in range(GQA)], axis=0)
                s = lax.dot(qg, kbt,
                            preferred_element_type=jnp.float32)
                m = jnp.max(s, axis=1, keepdims=True)
                p = jnp.exp(s - m)
                l = jnp.sum(p, axis=1, keepdims=True)
                o = lax.dot_general(
                    p.astype(jnp.bfloat16), vbt, (((1,), (1,)), ((), ())),
                    preferred_element_type=jnp.float32)
                o = o / l
                heads.extend(o[hh * Sq:(hh + 1) * Sq, :] for hh in range(GQA))
            batch_rows.append(jnp.concatenate(heads, axis=1))
        attn = jnp.concatenate(batch_rows, axis=0)

        cp_wo.wait()
        wo = wo_vmem[:].astype(jnp.bfloat16)
        partial = lax.dot(attn.astype(jnp.bfloat16), wo,
                          preferred_element_type=jnp.float32)

        comm_ref[0, :, :] = partial.astype(jnp.bfloat16)
        rdmas = []
        for d in range(1, N_DEV):
            peer = lax.rem(my_i + d, N_DEV)
            slot = N_DEV - d
            rdma = pltpu.make_async_remote_copy(
                src_ref=comm_ref.at[0],
                dst_ref=comm_ref.at[slot],
                send_sem=send_sems.at[d - 1],
                recv_sem=recv_sems.at[slot - 1],
                device_id=(peer,),
                device_id_type=pl.DeviceIdType.MESH,
            )
            rdma.start()
            rdmas.append(rdma)

        acc = partial
        for s in range(1, N_DEV):
            rdmas[N_DEV - 1 - s].wait_recv()
            acc = acc + comm_ref[s, :, :].astype(jnp.float32)

        for rdma in rdmas:
            rdma.wait_send()

        out_ref[:] = acc.astype(jnp.bfloat16).reshape(B, Sq, Dout)

    return pl.pallas_call(
        body,
        out_shape=jax.ShapeDtypeStruct((B, Sq, Dout), jnp.bfloat16),
        in_specs=[
            pl.BlockSpec(memory_space=pltpu.VMEM),
            pl.BlockSpec(memory_space=pl.ANY),
            pl.BlockSpec(memory_space=pl.ANY),
            pl.BlockSpec(memory_space=pl.ANY),
            pl.BlockSpec(memory_space=pl.ANY),
        ],
        out_specs=pl.BlockSpec(memory_space=pltpu.VMEM),
        scratch_shapes=[
            pltpu.VMEM((D, Dq), jnp.float32),
            pltpu.VMEM((Dq, Dout), jnp.float32),
            pltpu.VMEM((B, Hkv_local, Dh, Skv), jnp.float32),
            pltpu.VMEM((B, Hkv_local, Dh, Skv), jnp.float32),
            pltpu.SemaphoreType.DMA((4,)),
            pltpu.VMEM((N_DEV, M, Dout), jnp.bfloat16),
            pltpu.SemaphoreType.DMA((N_DEV - 1,)),
            pltpu.SemaphoreType.DMA((N_DEV - 1,)),
        ],
        compiler_params=pltpu.CompilerParams(collective_id=0),
    )(x, Wq, Wo, Kt, Vt)


# device time: 15032 ns/iter; 1.2985x vs baseline; 1.2985x over previous
import jax
import jax.numpy as jnp
from jax import lax
from jax.experimental import pallas as pl
from jax.experimental.pallas import tpu as pltpu

N_DEV = 4


def kernel(x, Wq, Wo, K_ext, V_ext):
    B, Sq, D = x.shape
    Dq = Wq.shape[1]
    Dh = K_ext.shape[3]
    Skv = K_ext.shape[1]
    Hq_local = Dq // Dh
    GQA = 4
    Hkv_local = Hq_local // GQA
    Dout = Wo.shape[1]
    M = B * Sq

    my_idx = lax.axis_index("i")
    Kt = lax.dynamic_slice_in_dim(
        jnp.transpose(K_ext, (0, 2, 3, 1)), 2 * my_idx, Hkv_local, axis=1)
    Vt = lax.dynamic_slice_in_dim(
        jnp.transpose(V_ext, (0, 2, 3, 1)), 2 * my_idx, Hkv_local, axis=1)

    def body(x_ref, wq_ref, wo_ref, kt_ref, vt_ref, out_ref,
             comm_ref, send_sems, recv_sems):
        my_i = lax.axis_index("i")

        barrier_sem = pltpu.get_barrier_semaphore()
        for d in range(1, N_DEV):
            peer = lax.rem(my_i + d, N_DEV)
            pl.semaphore_signal(
                barrier_sem, inc=1,
                device_id=(peer,), device_id_type=pl.DeviceIdType.MESH,
            )

        xv = x_ref[:].reshape(M, D).astype(jnp.bfloat16)
        wq = wq_ref[:].astype(jnp.bfloat16)
        q2 = (lax.dot(xv, wq, preferred_element_type=jnp.float32)
              * 0.125).astype(jnp.bfloat16)
        wo = wo_ref[:].astype(jnp.bfloat16)

        partials = []
        rdmas = {}
        for b in range(B):
            qb = q2[b * Sq:(b + 1) * Sq, :]
            heads = []
            for g in range(Hkv_local):
                kbt = kt_ref[b, g].astype(jnp.bfloat16)
                vbt = vt_ref[b, g].astype(jnp.bfloat16)
                qg = jnp.concatenate(
                    [qb[:, (g * GQA + hh) * Dh:(g * GQA + hh + 1) * Dh]
                     for hh in range(GQA)], axis=0)
                s = lax.dot(qg, kbt,
                            preferred_element_type=jnp.float32)
                m = jnp.max(s, axis=1, keepdims=True)
                p = jnp.exp(s - m)
                l = jnp.sum(p, axis=1, keepdims=True)
                o = lax.dot_general(
                    p.astype(jnp.bfloat16), vbt, (((1,), (1,)), ((), ())),
                    preferred_element_type=jnp.float32)
                o = o / l
                heads.extend(o[hh * Sq:(hh + 1) * Sq, :] for hh in range(GQA))
            attn_b = jnp.concatenate(heads, axis=1)
            partial_b = lax.dot(attn_b.astype(jnp.bfloat16), wo,
                                preferred_element_type=jnp.float32)
            partials.append(partial_b)

            comm_ref[0, pl.ds(b * Sq, Sq), :] = partial_b.astype(jnp.bfloat16)
            if b == 0:
                pl.semaphore_wait(barrier_sem, N_DEV - 1)
            for d in range(1, N_DEV):
                peer = lax.rem(my_i + d, N_DEV)
                slot = N_DEV - d
                rdma = pltpu.make_async_remote_copy(
                    src_ref=comm_ref.at[0, pl.ds(b * Sq, Sq)],
                    dst_ref=comm_ref.at[slot, pl.ds(b * Sq, Sq)],
                    send_sem=send_sems.at[d - 1, b],
                    recv_sem=recv_sems.at[slot - 1, b],
                    device_id=(peer,),
                    device_id_type=pl.DeviceIdType.MESH,
                )
                rdma.start()
                rdmas[(slot, b)] = rdma

        for b in range(B):
            acc = partials[b]
            for s in range(1, N_DEV):
                rdmas[(s, b)].wait_recv()
                acc = acc + comm_ref[s, pl.ds(b * Sq, Sq), :].astype(
                    jnp.float32)
            out_ref[b] = acc.astype(jnp.bfloat16)

        for rdma in rdmas.values():
            rdma.wait_send()

    return pl.pallas_call(
        body,
        out_shape=jax.ShapeDtypeStruct((B, Sq, Dout), jnp.bfloat16),
        in_specs=[pl.BlockSpec(memory_space=pltpu.VMEM)] * 5,
        out_specs=pl.BlockSpec(memory_space=pltpu.VMEM),
        scratch_shapes=[
            pltpu.VMEM((N_DEV, M, Dout), jnp.bfloat16),
            pltpu.SemaphoreType.DMA((N_DEV - 1, B)),
            pltpu.SemaphoreType.DMA((N_DEV - 1, B)),
        ],
        compiler_params=pltpu.CompilerParams(collective_id=0),
    )(x, Wq, Wo, Kt, Vt)


# device time: 14223 ns/iter; 1.3724x vs baseline; 1.0569x over previous
import jax
import jax.numpy as jnp
from jax import lax
from jax.experimental import pallas as pl
from jax.experimental.pallas import tpu as pltpu

N_DEV = 4


def kernel(x, Wq, Wo, K_ext, V_ext):
    B, Sq, D = x.shape
    Dq = Wq.shape[1]
    Dh = K_ext.shape[3]
    Skv = K_ext.shape[1]
    Hq_local = Dq // Dh
    GQA = 4
    Hkv_local = Hq_local // GQA
    Dout = Wo.shape[1]
    M = B * Sq

    my_idx = lax.axis_index("i")
    xb = x.astype(jnp.bfloat16)
    Wqb = (Wq * 0.125).astype(jnp.bfloat16)
    Wob = Wo.astype(jnp.bfloat16)
    Kt = lax.dynamic_slice_in_dim(
        jnp.transpose(K_ext, (0, 2, 3, 1)), 2 * my_idx, Hkv_local,
        axis=1).astype(jnp.bfloat16)
    Vt = lax.dynamic_slice_in_dim(
        jnp.transpose(V_ext, (0, 2, 3, 1)), 2 * my_idx, Hkv_local,
        axis=1).astype(jnp.bfloat16)

    def body(x_ref, wq_ref, wo_ref, kt_ref, vt_ref, out_ref,
             comm_ref, obuf_ref, send_sems, recv_sems, out_sems):
        my_i = lax.axis_index("i")

        barrier_sem = pltpu.get_barrier_semaphore()
        for d in range(1, N_DEV):
            peer = lax.rem(my_i + d, N_DEV)
            pl.semaphore_signal(
                barrier_sem, inc=1,
                device_id=(peer,), device_id_type=pl.DeviceIdType.MESH,
            )

        xv = x_ref[:].reshape(M, D)
        q2 = lax.dot(xv, wq_ref[:],
                     preferred_element_type=jnp.float32
                     ).astype(jnp.bfloat16)
        wo = wo_ref[:]

        partials = []
        rdmas = {}
        for b in range(B):
            qb = q2[b * Sq:(b + 1) * Sq, :]
            heads = []
            for g in range(Hkv_local):
                kbt = kt_ref[b, g]
                vbt = vt_ref[b, g]
                qg = jnp.concatenate(
                    [qb[:, (g * GQA + hh) * Dh:(g * GQA + hh + 1) * Dh]
                     for hh in range(GQA)], axis=0)
                s = lax.dot(qg, kbt,
                            preferred_element_type=jnp.float32)
                m = jnp.max(s, axis=1, keepdims=True)
                p = jnp.exp(s - m)
                l = jnp.sum(p, axis=1, keepdims=True)
                pb = (p * (1.0 / l)).astype(jnp.bfloat16)
                o = lax.dot_general(
                    pb, vbt, (((1,), (1,)), ((), ())),
                    preferred_element_type=jnp.float32
                    ).astype(jnp.bfloat16)
                heads.extend(o[hh * Sq:(hh + 1) * Sq, :] for hh in range(GQA))
            attn_b = jnp.concatenate(heads, axis=1)
            partial_b = lax.dot(attn_b, wo,
                                preferred_element_type=jnp.float32)
            partials.append(partial_b)

            comm_ref[0, pl.ds(b * Sq, Sq), :] = partial_b.astype(jnp.bfloat16)
            if b == 0:
                pl.semaphore_wait(barrier_sem, N_DEV - 1)
            for d in range(1, N_DEV):
                peer = lax.rem(my_i + d, N_DEV)
                slot = N_DEV - d
                rdma = pltpu.make_async_remote_copy(
                    src_ref=comm_ref.at[0, pl.ds(b * Sq, Sq)],
                    dst_ref=comm_ref.at[slot, pl.ds(b * Sq, Sq)],
                    send_sem=send_sems.at[d - 1, b],
                    recv_sem=recv_sems.at[slot - 1, b],
                    device_id=(peer,),
                    device_id_type=pl.DeviceIdType.MESH,
                )
                rdma.start()
                rdmas[(slot, b)] = rdma

        out_dmas = []
        for b in range(B):
            acc = partials[b]
            for s in range(1, N_DEV):
                rdmas[(s, b)].wait_recv()
                acc = acc + comm_ref[s, pl.ds(b * Sq, Sq), :].astype(
                    jnp.float32)
            obuf_ref[pl.ds(b * Sq, Sq), :] = acc.astype(jnp.bfloat16)
            dma = pltpu.make_async_copy(
                obuf_ref.at[pl.ds(b * Sq, Sq)], out_ref.at[b],
                out_sems.at[b])
            dma.start()
            out_dmas.append(dma)

        for dma in out_dmas:
            dma.wait()
        for rdma in rdmas.values():
            rdma.wait_send()

    return pl.pallas_call(
        body,
        out_shape=jax.ShapeDtypeStruct((B, Sq, Dout), jnp.bfloat16),
        in_specs=[pl.BlockSpec(memory_space=pltpu.VMEM)] * 5,
        out_specs=pl.BlockSpec(memory_space=pl.ANY),
        scratch_shapes=[
            pltpu.VMEM((N_DEV, M, Dout), jnp.bfloat16),
            pltpu.VMEM((M, Dout), jnp.bfloat16),
            pltpu.SemaphoreType.DMA((N_DEV - 1, B)),
            pltpu.SemaphoreType.DMA((N_DEV - 1, B)),
            pltpu.SemaphoreType.DMA((B,)),
        ],
        compiler_params=pltpu.CompilerParams(collective_id=0),
    )(xb, Wqb, Wob, Kt, Vt)
